# baseline (speedup 1.0000x reference)
import jax
import jax.numpy as jnp
from jax.experimental import pallas as pl
from jax.experimental.pallas import tpu as pltpu

_N = 4096
_L = 16384
_BR = 248
_NBUF = 4
_NFULL = _N // _BR
_TAIL = _N - _NFULL * _BR
_NCHUNK = _NFULL + 1


def _chunk_rows(c):
    return (_TAIL if c == _NFULL else _BR), c * _BR


def _ring_kernel(x_hbm, sel_ref, o_hbm, bufs, in_sems, out_sems):
    def start_in(c):
        rows, base = _chunk_rows(c)
        b = c % _NBUF
        pltpu.make_async_copy(
            x_hbm.at[pl.ds(base, rows)],
            bufs.at[b, pl.ds(0, rows)],
            in_sems.at[b],
        ).start()

    def wait_in(c):
        rows, base = _chunk_rows(c)
        b = c % _NBUF
        pltpu.make_async_copy(
            x_hbm.at[pl.ds(base, rows)],
            bufs.at[b, pl.ds(0, rows)],
            in_sems.at[b],
        ).wait()

    def start_out(c):
        rows, base = _chunk_rows(c)
        b = c % _NBUF
        pltpu.make_async_copy(
            bufs.at[b, pl.ds(0, rows)],
            o_hbm.at[pl.ds(base, rows)],
            out_sems.at[b],
        ).start()

    def wait_out(c):
        rows, base = _chunk_rows(c)
        b = c % _NBUF
        pltpu.make_async_copy(
            bufs.at[b, pl.ds(0, rows)],
            o_hbm.at[pl.ds(base, rows)],
            out_sems.at[b],
        ).wait()

    for c in range(min(2, _NCHUNK)):
        start_in(c)

    for c in range(_NCHUNK):
        rows, base = _chunk_rows(c)
        b = c % _NBUF
        wait_in(c)
        ids = base + jax.lax.broadcasted_iota(jnp.int32, (rows, 1), 0)
        hit = jnp.any(ids == sel_ref[...], axis=1, keepdims=True)
        sign = jnp.where(hit, -1.0, 1.0)
        bufs[b, pl.ds(0, rows)] = bufs[b, pl.ds(0, rows)] * sign
        start_out(c)
        nxt = c + 2
        if nxt < _NCHUNK:
            if nxt >= _NBUF:
                wait_out(nxt - _NBUF)
            start_in(nxt)

    for c in range(max(_NCHUNK - _NBUF, 0), _NCHUNK):
        wait_out(c)


def kernel(data, selection):
    n, l = data.shape
    sel2d = selection.astype(jnp.int32).reshape(1, -1)
    return pl.pallas_call(
        _ring_kernel,
        in_specs=[
            pl.BlockSpec(memory_space=pl.ANY),
            pl.BlockSpec((1, 64), lambda: (0, 0)),
        ],
        out_specs=pl.BlockSpec(memory_space=pl.ANY),
        out_shape=jax.ShapeDtypeStruct((n, l), data.dtype),
        scratch_shapes=[
            pltpu.VMEM((_NBUF, _BR, _L), jnp.float32),
            pltpu.SemaphoreType.DMA((_NBUF,)),
            pltpu.SemaphoreType.DMA((_NBUF,)),
        ],
        compiler_params=pltpu.CompilerParams(
            vmem_limit_bytes=128 * 1024 * 1024,
        ),
    )(data, sel2d)


# speedup vs baseline: 1.0048x; 1.0048x over previous
import jax
import jax.numpy as jnp
from jax.experimental import pallas as pl
from jax.experimental.pallas import tpu as pltpu

_BR = 248


def _flip_kernel(x_ref, sel_ref, o_ref):
    i = pl.program_id(0)
    rows = i * _BR + jax.lax.broadcasted_iota(jnp.int32, (_BR, 1), 0)
    hit = jnp.any(rows == sel_ref[...], axis=1, keepdims=True)
    sign = jnp.where(hit, -1.0, 1.0).astype(x_ref.dtype)
    o_ref[...] = x_ref[...] * sign


def kernel(data, selection):
    n, l = data.shape
    sel2d = selection.astype(jnp.int32).reshape(1, -1)
    return pl.pallas_call(
        _flip_kernel,
        grid=(pl.cdiv(n, _BR),),
        in_specs=[
            pl.BlockSpec((_BR, l), lambda i: (i, 0)),
            pl.BlockSpec(sel2d.shape, lambda i: (0, 0)),
        ],
        out_specs=pl.BlockSpec((_BR, l), lambda i: (i, 0)),
        out_shape=jax.ShapeDtypeStruct((n, l), data.dtype),
        compiler_params=pltpu.CompilerParams(
            dimension_semantics=("parallel",),
            vmem_limit_bytes=128 * 1024 * 1024,
        ),
    )(data, sel2d)
